# Initial kernel scaffold; baseline (speedup 1.0000x reference)
import jax
import jax.numpy as jnp
from jax.experimental import pallas as pl


def kernel(features, edge_index, W_red, b_red, conv_weight_0, conv_bias_0, conv_weight_1, conv_bias_1, W_cls, b_cls):
    raise NotImplementedError("write your pallas kernel here")


# speedup vs baseline: 5.0618x; 5.0618x over previous
import functools

import jax
import jax.numpy as jnp
from jax import lax
from jax.experimental import pallas as pl
from jax.experimental.pallas import tpu as pltpu
from jax.experimental.pallas import tpu_sc as plsc

N = 50000
N_OUT = 40000
E_EDGES = 800000
D_IN = 128
H_DIM = 64
HH = 32
C_CLS = 50

ROW_BLK = 1000
N_SUBC = 16
NPAD = 50048
TILE_ROWS = NPAD // N_SUBC
ZCHUNK = 184
EB = 80
TILE_EDGES = E_EDGES // N_SUBC
NBATCH = TILE_EDGES // EB

_sc_mesh = plsc.VectorSubcoreMesh(core_axis_name="c", subcore_axis_name="s")


@functools.partial(
    pl.kernel,
    mesh=_sc_mesh,
    compiler_params=pltpu.CompilerParams(use_tc_tiling_on_sc=False),
    out_type=jax.ShapeDtypeStruct((NPAD, 16), jnp.float32),
    scratch_types=[
        pltpu.VMEM((EB,), jnp.int32),
        pltpu.VMEM((EB, 16), jnp.float32),
        pltpu.VMEM((ZCHUNK, 16), jnp.float32),
        pltpu.VMEM_SHARED((NPAD, 16), jnp.float32),
    ],
)
def _deg_sc(dst, out_deg, dst_v, ones_v, zbuf, acc):
    c = lax.axis_index("c")
    s = lax.axis_index("s")

    @pl.when(c == 0)
    def _():
        def fill_ones(i, carry):
            ones_v[i, :] = jnp.ones((16,), jnp.float32)
            return carry
        lax.fori_loop(0, EB, fill_ones, 0)

        def fill_zero(i, carry):
            zbuf[i, :] = jnp.zeros((16,), jnp.float32)
            return carry
        lax.fori_loop(0, ZCHUNK, fill_zero, 0)

        row0 = s * TILE_ROWS
        def zero_acc(j, carry):
            pltpu.sync_copy(zbuf, acc.at[pl.ds(row0 + j * ZCHUNK, ZCHUNK)])
            return carry
        lax.fori_loop(0, TILE_ROWS // ZCHUNK, zero_acc, 0)
        plsc.subcore_barrier()

        base = s * TILE_EDGES
        def body(i, carry):
            pltpu.sync_copy(dst.at[pl.ds(base + i * EB, EB)], dst_v)
            pltpu.sync_copy(ones_v, acc.at[dst_v], add=True)
            return carry
        lax.fori_loop(0, NBATCH, body, 0)
        plsc.subcore_barrier()

        pltpu.sync_copy(acc.at[pl.ds(row0, TILE_ROWS)],
                        out_deg.at[pl.ds(row0, TILE_ROWS)])


@functools.partial(
    pl.kernel,
    mesh=_sc_mesh,
    compiler_params=pltpu.CompilerParams(use_tc_tiling_on_sc=False),
    out_type=[jax.ShapeDtypeStruct((NPAD, HH), jnp.float32),
              jax.ShapeDtypeStruct((NPAD, HH), jnp.float32)],
    scratch_types=[
        pltpu.VMEM((EB,), jnp.int32),
        pltpu.VMEM((EB,), jnp.int32),
        pltpu.VMEM((EB, HH), jnp.float32),
        pltpu.VMEM((ZCHUNK, HH), jnp.float32),
        pltpu.VMEM_SHARED((NPAD, HH), jnp.float32),
        pltpu.SemaphoreType.DMA,
    ],
)
def _spmm_sc(ya, yb, src, dst, out_a, out_b, src_v, dst_v, rows_v, zbuf, acc, sem):
    c = lax.axis_index("c")
    s = lax.axis_index("s")

    def fill_zero(i, carry):
        zbuf[i, pl.ds(0, 16)] = jnp.zeros((16,), jnp.float32)
        zbuf[i, pl.ds(16, 16)] = jnp.zeros((16,), jnp.float32)
        return carry
    lax.fori_loop(0, ZCHUNK, fill_zero, 0)

    row0 = s * TILE_ROWS
    def zero_acc(j, carry):
        pltpu.sync_copy(zbuf, acc.at[pl.ds(row0 + j * ZCHUNK, ZCHUNK)])
        return carry
    lax.fori_loop(0, TILE_ROWS // ZCHUNK, zero_acc, 0)
    plsc.subcore_barrier()

    def edge_pass(y_hbm):
        base = s * TILE_EDGES
        def body(i, carry):
            off = base + i * EB
            pltpu.sync_copy(src.at[pl.ds(off, EB)], src_v)
            pltpu.sync_copy(dst.at[pl.ds(off, EB)], dst_v)
            pltpu.async_copy(y_hbm.at[src_v], rows_v, sem).wait()
            pltpu.sync_copy(rows_v, acc.at[dst_v], add=True)
            return carry
        lax.fori_loop(0, NBATCH, body, 0)

    pl.when(c == 0)(lambda: edge_pass(ya))
    pl.when(c == 1)(lambda: edge_pass(yb))
    plsc.subcore_barrier()

    pl.when(c == 0)(lambda: pltpu.sync_copy(acc.at[pl.ds(row0, TILE_ROWS)],
                                            out_a.at[pl.ds(row0, TILE_ROWS)]))
    pl.when(c == 1)(lambda: pltpu.sync_copy(acc.at[pl.ds(row0, TILE_ROWS)],
                                            out_b.at[pl.ds(row0, TILE_ROWS)]))


def _dense0_body(f_ref, wred_ref, bred_ref, w0_ref, b0_ref, ae_ref, ya_ref, yb_ref):
    ae = jnp.dot(f_ref[...], wred_ref[...],
                 preferred_element_type=jnp.float32) + bred_ref[...]
    y0 = jnp.dot(ae, w0_ref[...], preferred_element_type=jnp.float32) + b0_ref[...]
    ae_ref[...] = ae
    ya_ref[...] = y0[:, :HH]
    yb_ref[...] = y0[:, HH:]


def _dense0(features, W_red, b_red, W0, b0):
    return pl.pallas_call(
        _dense0_body,
        grid=(N // ROW_BLK,),
        in_specs=[
            pl.BlockSpec((ROW_BLK, D_IN), lambda i: (i, 0)),
            pl.BlockSpec((D_IN, H_DIM), lambda i: (0, 0)),
            pl.BlockSpec((1, H_DIM), lambda i: (0, 0)),
            pl.BlockSpec((H_DIM, H_DIM), lambda i: (0, 0)),
            pl.BlockSpec((1, H_DIM), lambda i: (0, 0)),
        ],
        out_specs=[
            pl.BlockSpec((ROW_BLK, H_DIM), lambda i: (i, 0)),
            pl.BlockSpec((ROW_BLK, HH), lambda i: (i, 0)),
            pl.BlockSpec((ROW_BLK, HH), lambda i: (i, 0)),
        ],
        out_shape=[
            jax.ShapeDtypeStruct((N, H_DIM), jnp.float32),
            jax.ShapeDtypeStruct((N, HH), jnp.float32),
            jax.ShapeDtypeStruct((N, HH), jnp.float32),
        ],
    )(features, W_red, b_red.reshape(1, -1), W0, b0.reshape(1, -1))


def _dense1_body(sa_ref, sb_ref, deg_ref, w1_ref, b1_ref, x1_ref, ya_ref, yb_ref):
    inv = 1.0 / jnp.maximum(deg_ref[:, 0:1], 1.0)
    x1 = jnp.concatenate([sa_ref[...] * inv, sb_ref[...] * inv], axis=1)
    y1 = jnp.dot(x1, w1_ref[...], preferred_element_type=jnp.float32) + b1_ref[...]
    x1_ref[...] = x1
    ya_ref[...] = y1[:, :HH]
    yb_ref[...] = y1[:, HH:]


def _dense1(s0a, s0b, deg16, W1, b1):
    return pl.pallas_call(
        _dense1_body,
        grid=(N // ROW_BLK,),
        in_specs=[
            pl.BlockSpec((ROW_BLK, HH), lambda i: (i, 0)),
            pl.BlockSpec((ROW_BLK, HH), lambda i: (i, 0)),
            pl.BlockSpec((ROW_BLK, 16), lambda i: (i, 0)),
            pl.BlockSpec((H_DIM, H_DIM), lambda i: (0, 0)),
            pl.BlockSpec((1, H_DIM), lambda i: (0, 0)),
        ],
        out_specs=[
            pl.BlockSpec((ROW_BLK, H_DIM), lambda i: (i, 0)),
            pl.BlockSpec((ROW_BLK, HH), lambda i: (i, 0)),
            pl.BlockSpec((ROW_BLK, HH), lambda i: (i, 0)),
        ],
        out_shape=[
            jax.ShapeDtypeStruct((N, H_DIM), jnp.float32),
            jax.ShapeDtypeStruct((N, HH), jnp.float32),
            jax.ShapeDtypeStruct((N, HH), jnp.float32),
        ],
    )(s0a, s0b, deg16, W1, b1.reshape(1, -1))


def _final_body(ae_ref, x1_ref, sa_ref, sb_ref, deg_ref, wc_ref, bc_ref, out_ref):
    inv = 1.0 / jnp.maximum(deg_ref[:, 0:1], 1.0)
    x2 = jnp.concatenate([sa_ref[...] * inv, sb_ref[...] * inv], axis=1)
    m = (ae_ref[...] + x1_ref[...] + x2) * (1.0 / 3.0)
    z = jnp.dot(m, wc_ref[...], preferred_element_type=jnp.float32) + bc_ref[...]
    zmax = jnp.max(z, axis=1, keepdims=True)
    lse = jnp.log(jnp.sum(jnp.exp(z - zmax), axis=1, keepdims=True)) + zmax
    out_ref[...] = z - lse


def _final(all_emb, x1, s1a, s1b, deg16, W_cls, b_cls):
    return pl.pallas_call(
        _final_body,
        grid=(N_OUT // ROW_BLK,),
        in_specs=[
            pl.BlockSpec((ROW_BLK, H_DIM), lambda i: (i, 0)),
            pl.BlockSpec((ROW_BLK, H_DIM), lambda i: (i, 0)),
            pl.BlockSpec((ROW_BLK, HH), lambda i: (i, 0)),
            pl.BlockSpec((ROW_BLK, HH), lambda i: (i, 0)),
            pl.BlockSpec((ROW_BLK, 16), lambda i: (i, 0)),
            pl.BlockSpec((H_DIM, C_CLS), lambda i: (0, 0)),
            pl.BlockSpec((1, C_CLS), lambda i: (0, 0)),
        ],
        out_specs=pl.BlockSpec((ROW_BLK, C_CLS), lambda i: (i, 0)),
        out_shape=jax.ShapeDtypeStruct((N_OUT, C_CLS), jnp.float32),
    )(all_emb, x1, s1a, s1b, deg16, W_cls, b_cls.reshape(1, -1))


def kernel(features, edge_index, W_red, b_red, conv_weight_0, conv_bias_0,
           conv_weight_1, conv_bias_1, W_cls, b_cls):
    src = edge_index[0]
    dst = edge_index[1]
    deg16 = _deg_sc(dst)
    all_emb, y0a, y0b = _dense0(features, W_red, b_red, conv_weight_0, conv_bias_0)
    s0a, s0b = _spmm_sc(y0a, y0b, src, dst)
    x1, y1a, y1b = _dense1(s0a, s0b, deg16, conv_weight_1, conv_bias_1)
    s1a, s1b = _spmm_sc(y1a, y1b, src, dst)
    return _final(all_emb, x1, s1a, s1b, deg16, W_cls, b_cls)


# speedup vs baseline: 13.2662x; 2.6208x over previous
import functools

import jax
import jax.numpy as jnp
from jax import lax
from jax.experimental import pallas as pl
from jax.experimental.pallas import tpu as pltpu
from jax.experimental.pallas import tpu_sc as plsc

N = 50000
N_OUT = 40000
E_EDGES = 800000
D_IN = 128
H_DIM = 64
HH = 32
C_CLS = 50

ROW_BLK = 1000
N_SUBC = 16
NPAD = 50048
TILE_ROWS = NPAD // N_SUBC
ZCHUNK = 184
EB = 80
TILE_EDGES = E_EDGES // N_SUBC
NBATCH = TILE_EDGES // EB
CH = 25
NCHUNK = NBATCH // CH

_sc_mesh = plsc.VectorSubcoreMesh(core_axis_name="c", subcore_axis_name="s")


@functools.partial(
    pl.kernel,
    mesh=_sc_mesh,
    compiler_params=pltpu.CompilerParams(use_tc_tiling_on_sc=False),
    out_type=jax.ShapeDtypeStruct((NPAD, 16), jnp.float32),
    scratch_types=[
        pltpu.VMEM((CH, EB), jnp.int32),
        pltpu.VMEM((EB, 16), jnp.float32),
        pltpu.VMEM((ZCHUNK, 16), jnp.float32),
        pltpu.VMEM_SHARED((NPAD, 16), jnp.float32),
        pltpu.SemaphoreType.DMA,
        pltpu.SemaphoreType.DMA,
    ],
)
def _deg_sc(dst2, out_deg, dbuf, ones_v, zbuf, acc, sem0, sem1):
    c = lax.axis_index("c")
    s = lax.axis_index("s")
    sems = (sem0, sem1)

    @pl.when(c == 0)
    def _():
        def fill_ones(i, carry):
            ones_v[i, :] = jnp.ones((16,), jnp.float32)
            return carry
        lax.fori_loop(0, EB, fill_ones, 0)

        def fill_zero(i, carry):
            zbuf[i, :] = jnp.zeros((16,), jnp.float32)
            return carry
        lax.fori_loop(0, ZCHUNK, fill_zero, 0)

        row0 = s * TILE_ROWS
        def zero_acc(j, carry):
            pltpu.sync_copy(zbuf, acc.at[pl.ds(row0 + j * ZCHUNK, ZCHUNK)])
            return carry
        lax.fori_loop(0, TILE_ROWS // ZCHUNK, zero_acc, 0)
        plsc.subcore_barrier()

        base = s * (NBATCH)
        def chunk(ci, carry):
            pltpu.sync_copy(dst2.at[pl.ds(base + ci * CH, CH)], dbuf)
            sc = {}
            for j in range(CH):
                sc[j] = pltpu.async_copy(ones_v, acc.at[dbuf.at[j]],
                                         sems[j & 1], add=True)
                if j >= 2:
                    sc[j - 2].wait()
            sc[CH - 2].wait()
            sc[CH - 1].wait()
            return carry
        lax.fori_loop(0, NCHUNK, chunk, 0)
        plsc.subcore_barrier()

        pltpu.sync_copy(acc.at[pl.ds(row0, TILE_ROWS)],
                        out_deg.at[pl.ds(row0, TILE_ROWS)])


@functools.partial(
    pl.kernel,
    mesh=_sc_mesh,
    compiler_params=pltpu.CompilerParams(use_tc_tiling_on_sc=False),
    out_type=[jax.ShapeDtypeStruct((NPAD, HH), jnp.float32),
              jax.ShapeDtypeStruct((NPAD, HH), jnp.float32)],
    scratch_types=[
        pltpu.VMEM((CH, EB), jnp.int32),
        pltpu.VMEM((CH, EB), jnp.int32),
        pltpu.VMEM((EB, HH), jnp.float32),
        pltpu.VMEM((EB, HH), jnp.float32),
        pltpu.VMEM((ZCHUNK, HH), jnp.float32),
        pltpu.VMEM_SHARED((NPAD, HH), jnp.float32),
        pltpu.SemaphoreType.DMA,
        pltpu.SemaphoreType.DMA,
        pltpu.SemaphoreType.DMA,
        pltpu.SemaphoreType.DMA,
    ],
)
def _spmm_sc(ya, yb, src2, dst2, out_a, out_b, sbuf, dbuf, rows0, rows1,
             zbuf, acc, gsem0, gsem1, ssem0, ssem1):
    c = lax.axis_index("c")
    s = lax.axis_index("s")
    rows = (rows0, rows1)
    gsems = (gsem0, gsem1)
    ssems = (ssem0, ssem1)

    def fill_zero(i, carry):
        zbuf[i, pl.ds(0, 16)] = jnp.zeros((16,), jnp.float32)
        zbuf[i, pl.ds(16, 16)] = jnp.zeros((16,), jnp.float32)
        return carry
    lax.fori_loop(0, ZCHUNK, fill_zero, 0)

    row0 = s * TILE_ROWS
    def zero_acc(j, carry):
        pltpu.sync_copy(zbuf, acc.at[pl.ds(row0 + j * ZCHUNK, ZCHUNK)])
        return carry
    lax.fori_loop(0, TILE_ROWS // ZCHUNK, zero_acc, 0)
    plsc.subcore_barrier()

    def edge_pass(y_hbm):
        base = s * NBATCH
        def chunk(ci, carry):
            pltpu.sync_copy(src2.at[pl.ds(base + ci * CH, CH)], sbuf)
            pltpu.sync_copy(dst2.at[pl.ds(base + ci * CH, CH)], dbuf)
            g, sc = {}, {}
            for j in range(CH):
                b = j & 1
                if j >= 2:
                    sc[j - 2].wait()
                g[j] = pltpu.async_copy(y_hbm.at[sbuf.at[j]], rows[b], gsems[b])
                if j >= 1:
                    g[j - 1].wait()
                    sc[j - 1] = pltpu.async_copy(
                        rows[(j - 1) & 1], acc.at[dbuf.at[j - 1]],
                        ssems[(j - 1) & 1], add=True)
            g[CH - 1].wait()
            sc[CH - 1] = pltpu.async_copy(
                rows[(CH - 1) & 1], acc.at[dbuf.at[CH - 1]],
                ssems[(CH - 1) & 1], add=True)
            sc[CH - 2].wait()
            sc[CH - 1].wait()
            return carry
        lax.fori_loop(0, NCHUNK, chunk, 0)

    pl.when(c == 0)(lambda: edge_pass(ya))
    pl.when(c == 1)(lambda: edge_pass(yb))
    plsc.subcore_barrier()

    pl.when(c == 0)(lambda: pltpu.sync_copy(acc.at[pl.ds(row0, TILE_ROWS)],
                                            out_a.at[pl.ds(row0, TILE_ROWS)]))
    pl.when(c == 1)(lambda: pltpu.sync_copy(acc.at[pl.ds(row0, TILE_ROWS)],
                                            out_b.at[pl.ds(row0, TILE_ROWS)]))


def _dense0_body(f_ref, wred_ref, bred_ref, w0_ref, b0_ref, ae_ref, ya_ref, yb_ref):
    ae = jnp.dot(f_ref[...], wred_ref[...],
                 preferred_element_type=jnp.float32) + bred_ref[...]
    y0 = jnp.dot(ae, w0_ref[...], preferred_element_type=jnp.float32) + b0_ref[...]
    ae_ref[...] = ae
    ya_ref[...] = y0[:, :HH]
    yb_ref[...] = y0[:, HH:]


def _dense0(features, W_red, b_red, W0, b0):
    return pl.pallas_call(
        _dense0_body,
        grid=(N // ROW_BLK,),
        in_specs=[
            pl.BlockSpec((ROW_BLK, D_IN), lambda i: (i, 0)),
            pl.BlockSpec((D_IN, H_DIM), lambda i: (0, 0)),
            pl.BlockSpec((1, H_DIM), lambda i: (0, 0)),
            pl.BlockSpec((H_DIM, H_DIM), lambda i: (0, 0)),
            pl.BlockSpec((1, H_DIM), lambda i: (0, 0)),
        ],
        out_specs=[
            pl.BlockSpec((ROW_BLK, H_DIM), lambda i: (i, 0)),
            pl.BlockSpec((ROW_BLK, HH), lambda i: (i, 0)),
            pl.BlockSpec((ROW_BLK, HH), lambda i: (i, 0)),
        ],
        out_shape=[
            jax.ShapeDtypeStruct((N, H_DIM), jnp.float32),
            jax.ShapeDtypeStruct((N, HH), jnp.float32),
            jax.ShapeDtypeStruct((N, HH), jnp.float32),
        ],
    )(features, W_red, b_red.reshape(1, -1), W0, b0.reshape(1, -1))


def _dense1_body(sa_ref, sb_ref, deg_ref, w1_ref, b1_ref, x1_ref, ya_ref, yb_ref):
    inv = 1.0 / jnp.maximum(deg_ref[:, 0:1], 1.0)
    x1 = jnp.concatenate([sa_ref[...] * inv, sb_ref[...] * inv], axis=1)
    y1 = jnp.dot(x1, w1_ref[...], preferred_element_type=jnp.float32) + b1_ref[...]
    x1_ref[...] = x1
    ya_ref[...] = y1[:, :HH]
    yb_ref[...] = y1[:, HH:]


def _dense1(s0a, s0b, deg16, W1, b1):
    return pl.pallas_call(
        _dense1_body,
        grid=(N // ROW_BLK,),
        in_specs=[
            pl.BlockSpec((ROW_BLK, HH), lambda i: (i, 0)),
            pl.BlockSpec((ROW_BLK, HH), lambda i: (i, 0)),
            pl.BlockSpec((ROW_BLK, 16), lambda i: (i, 0)),
            pl.BlockSpec((H_DIM, H_DIM), lambda i: (0, 0)),
            pl.BlockSpec((1, H_DIM), lambda i: (0, 0)),
        ],
        out_specs=[
            pl.BlockSpec((ROW_BLK, H_DIM), lambda i: (i, 0)),
            pl.BlockSpec((ROW_BLK, HH), lambda i: (i, 0)),
            pl.BlockSpec((ROW_BLK, HH), lambda i: (i, 0)),
        ],
        out_shape=[
            jax.ShapeDtypeStruct((N, H_DIM), jnp.float32),
            jax.ShapeDtypeStruct((N, HH), jnp.float32),
            jax.ShapeDtypeStruct((N, HH), jnp.float32),
        ],
    )(s0a, s0b, deg16, W1, b1.reshape(1, -1))


def _final_body(ae_ref, x1_ref, sa_ref, sb_ref, deg_ref, wc_ref, bc_ref, out_ref):
    inv = 1.0 / jnp.maximum(deg_ref[:, 0:1], 1.0)
    x2 = jnp.concatenate([sa_ref[...] * inv, sb_ref[...] * inv], axis=1)
    m = (ae_ref[...] + x1_ref[...] + x2) * (1.0 / 3.0)
    z = jnp.dot(m, wc_ref[...], preferred_element_type=jnp.float32) + bc_ref[...]
    zmax = jnp.max(z, axis=1, keepdims=True)
    lse = jnp.log(jnp.sum(jnp.exp(z - zmax), axis=1, keepdims=True)) + zmax
    out_ref[...] = z - lse


def _final(all_emb, x1, s1a, s1b, deg16, W_cls, b_cls):
    return pl.pallas_call(
        _final_body,
        grid=(N_OUT // ROW_BLK,),
        in_specs=[
            pl.BlockSpec((ROW_BLK, H_DIM), lambda i: (i, 0)),
            pl.BlockSpec((ROW_BLK, H_DIM), lambda i: (i, 0)),
            pl.BlockSpec((ROW_BLK, HH), lambda i: (i, 0)),
            pl.BlockSpec((ROW_BLK, HH), lambda i: (i, 0)),
            pl.BlockSpec((ROW_BLK, 16), lambda i: (i, 0)),
            pl.BlockSpec((H_DIM, C_CLS), lambda i: (0, 0)),
            pl.BlockSpec((1, C_CLS), lambda i: (0, 0)),
        ],
        out_specs=pl.BlockSpec((ROW_BLK, C_CLS), lambda i: (i, 0)),
        out_shape=jax.ShapeDtypeStruct((N_OUT, C_CLS), jnp.float32),
    )(all_emb, x1, s1a, s1b, deg16, W_cls, b_cls.reshape(1, -1))


def kernel(features, edge_index, W_red, b_red, conv_weight_0, conv_bias_0,
           conv_weight_1, conv_bias_1, W_cls, b_cls):
    src2 = edge_index[0].reshape(E_EDGES // EB, EB)
    dst2 = edge_index[1].reshape(E_EDGES // EB, EB)
    deg16 = _deg_sc(dst2)
    all_emb, y0a, y0b = _dense0(features, W_red, b_red, conv_weight_0, conv_bias_0)
    s0a, s0b = _spmm_sc(y0a, y0b, src2, dst2)
    x1, y1a, y1b = _dense1(s0a, s0b, deg16, conv_weight_1, conv_bias_1)
    s1a, s1b = _spmm_sc(y1a, y1b, src2, dst2)
    return _final(all_emb, x1, s1a, s1b, deg16, W_cls, b_cls)
